# baseline (speedup 1.0000x reference)
import functools

import jax
import jax.numpy as jnp
from jax import lax
from jax.experimental import pallas as pl
from jax.experimental.pallas import tpu as pltpu
from jax.experimental.pallas import tpu_sc as plsc

NC = 2
NS = 16
LANES = 16

GC = 128
TC_BLOCK = 1600


def _sc_gather(table, idx2, e_pad, dt):
  nw = NC * NS
  bpw = e_pad // nw
  steps = bpw // GC
  mesh = plsc.VectorSubcoreMesh(core_axis_name="c", subcore_axis_name="s")

  @functools.partial(
      pl.kernel,
      out_type=jax.ShapeDtypeStruct((e_pad, dt), jnp.float32),
      mesh=mesh,
      scratch_types=[
          pltpu.VMEM((bpw,), jnp.int32),
          pltpu.VMEM((4, GC, dt), jnp.float32),
          pltpu.SemaphoreType.DMA((4,)),
          pltpu.SemaphoreType.DMA((4,)),
      ],
  )
  def gk(table_hbm, idx_hbm, out_hbm, idxb, rows_v, gsem, wsem):
    wid = lax.axis_index("s") * NC + lax.axis_index("c")
    base = wid * bpw
    pltpu.sync_copy(idx_hbm.at[wid], idxb)
    g, w = {}, {}

    def start_gather(j):
      g[j] = pltpu.async_copy(table_hbm.at[idxb.at[pl.ds(j * GC, GC)]],
                              rows_v.at[j % 4], gsem.at[j % 4])

    for j in range(min(3, steps)):
      start_gather(j)
    for i in range(steps):
      g[i].wait()
      w[i] = pltpu.async_copy(
          rows_v.at[i % 4],
          out_hbm.at[pl.ds(pl.multiple_of(base + i * GC, GC), GC)],
          wsem.at[i % 4])
      nxt = i + 3
      if nxt < steps:
        if nxt - 4 >= 0:
          w[nxt - 4].wait()
        start_gather(nxt)
    for k in range(max(0, steps - 4), steps):
      w[k].wait()

  return gk(table, idx2)


def _sc_scatter_add(t, dst2, zeros, n_pad, dt):
  npc = n_pad // NC
  acc_rows = npc + 8
  e_pad = t.shape[0]
  steps = e_pad // NS // GC
  rows_per_tile = npc // NS
  mesh = plsc.VectorSubcoreMesh(core_axis_name="c", subcore_axis_name="s")

  @functools.partial(
      pl.kernel,
      out_type=jax.ShapeDtypeStruct((n_pad, dt), jnp.float32),
      mesh=mesh,
      scratch_types=[
          pltpu.VMEM((steps, GC), jnp.int32),
          pltpu.VMEM((steps, GC), jnp.int32),
          pltpu.VMEM((2, GC, dt), jnp.float32),
          pltpu.VMEM((rows_per_tile, dt), jnp.float32),
          pltpu.VMEM_SHARED((acc_rows, dt), jnp.float32),
          pltpu.SemaphoreType.DMA((2,)),
          pltpu.SemaphoreType.DMA((2,)),
      ],
  )
  def sk(t_hbm, dst_hbm, zeros_hbm, out_hbm, di, ri, rows_v, obuf, acc,
         lsem, asem):
    cid = lax.axis_index("c")
    sid = lax.axis_index("s")
    lo = cid * npc

    @pl.when(sid == 0)
    def _init():
      pltpu.sync_copy(zeros_hbm, acc)

    pltpu.sync_copy(dst_hbm.at[sid], di)

    def remap(r0, carry):
      for v in range(GC // LANES):
        dv = di[r0, pl.ds(v * LANES, LANES)]
        local = dv - lo
        ok = (local >= 0) & (local < npc)
        ri[r0, pl.ds(v * LANES, LANES)] = jnp.where(ok, local, npc)
      return carry

    lax.fori_loop(0, steps, remap, 0)
    plsc.subcore_barrier()

    base = sid * steps * GC
    ld, ad = {}, {}

    def start_load(j):
      ld[j] = pltpu.async_copy(
          t_hbm.at[pl.ds(pl.multiple_of(base + j * GC, GC), GC)],
          rows_v.at[j % 2], lsem.at[j % 2])

    for j in range(min(2, steps)):
      start_load(j)
    for i in range(steps):
      ld[i].wait()
      ad[i] = pltpu.async_copy(rows_v.at[i % 2], acc.at[ri.at[i]],
                               asem.at[i % 2], add=True)
      if i + 2 < steps:
        ad[i].wait()
        start_load(i + 2)
    for k in range(max(0, steps - 2), steps):
      ad[k].wait()

    plsc.subcore_barrier()
    pltpu.sync_copy(
        acc.at[pl.ds(pl.multiple_of(sid * rows_per_tile, 8), rows_per_tile)],
        obuf)
    pltpu.sync_copy(
        obuf,
        out_hbm.at[pl.ds(pl.multiple_of(lo + sid * rows_per_tile, 8),
                         rows_per_tile)])

  return sk(t, dst2, zeros)


def _tc_transform(xg, par, bond, kc, r, s):
  e_pad, dt = xg.shape
  e, bd = bond.shape
  d, gd = kc.shape

  gb = bd * d

  def body(x_ref, p_ref, c_ref, kc_ref, r_ref, s_ref, o_ref):
    xw = x_ref[...]
    p = p_ref[...]
    x = (xw[:, :d] + p * (xw[:, d:2 * d] - xw[:, :d])).astype(jnp.bfloat16)
    z = jnp.dot(x, kc_ref[...],
                preferred_element_type=jnp.float32).astype(jnp.bfloat16)
    ce = jnp.dot(c_ref[...], r_ref[...],
                 preferred_element_type=jnp.float32).astype(jnp.bfloat16)
    o_ref[:, :d] = (jnp.dot(z[:, :gb] * ce, s_ref[...],
                            preferred_element_type=jnp.float32)
                    + z[:, gb:gb + d].astype(jnp.float32))

  return pl.pallas_call(
      body,
      grid=(e // TC_BLOCK,),
      in_specs=[
          pl.BlockSpec((TC_BLOCK, dt), lambda i: (i, 0)),
          pl.BlockSpec((TC_BLOCK, 1), lambda i: (i, 0)),
          pl.BlockSpec((TC_BLOCK, bd), lambda i: (i, 0)),
          pl.BlockSpec((d, gd), lambda i: (0, 0)),
          pl.BlockSpec((bd, gb), lambda i: (0, 0)),
          pl.BlockSpec((gb, d), lambda i: (0, 0)),
      ],
      out_specs=pl.BlockSpec((TC_BLOCK, dt), lambda i: (i, 0)),
      out_shape=jax.ShapeDtypeStruct((e_pad, dt), jnp.float32),
      compiler_params=pltpu.CompilerParams(
          dimension_semantics=("arbitrary",)),
  )(xg, par, bond, kc, r, s)


def kernel(atom_features, bond_features, pair_indices, kernel, bias):
  n, d = atom_features.shape
  e, bd = bond_features.shape
  dp = 128
  groups = bd + 2
  gd = groups * d

  align = NC * NS * GC
  e_pad = -(-e // align) * align
  n_pad = -(-n // (NC * NS * 8)) * (NC * NS * 8)

  src = pair_indices[:, 1].astype(jnp.int32)
  dst = pair_indices[:, 0].astype(jnp.int32)
  pad = e_pad - e
  par = (src & 1).astype(jnp.float32).reshape(e, 1)
  src = jnp.concatenate([src, jnp.zeros((pad,), jnp.int32)])
  dst = jnp.concatenate([dst, jnp.full((pad,), n, jnp.int32)])
  nw = NC * NS
  src2 = (src >> 1).reshape(nw, e_pad // nw)
  dst2 = dst.reshape(NS, e_pad // NS // GC, GC)

  k3 = kernel.reshape(bd, d, d).transpose(2, 0, 1)
  b3 = bias.reshape(d, d).T[:, None, :]
  kc = jnp.concatenate([k3, b3, jnp.zeros((d, 1, d), jnp.float32)],
                       axis=1).reshape(d, gd).astype(jnp.bfloat16)
  r = jnp.repeat(jnp.eye(bd, dtype=jnp.float32), d, axis=1)
  s = jnp.tile(jnp.eye(d, dtype=jnp.bfloat16), (bd, 1))

  table = atom_features.reshape(n // 2, dp)
  gathered = _sc_gather(table, src2, e_pad, dp)
  t = _tc_transform(gathered, par, bond_features, kc, r, s)
  zeros = jnp.zeros((n_pad // NC + 8, dp), jnp.float32)
  out = _sc_scatter_add(t, dst2, zeros, n_pad, dp)
  return out[:n, :d]


# speedup vs baseline: 1.0707x; 1.0707x over previous
import functools

import jax
import jax.numpy as jnp
from jax import lax
from jax.experimental import pallas as pl
from jax.experimental.pallas import tpu as pltpu
from jax.experimental.pallas import tpu_sc as plsc

NC = 2
NS = 16
LANES = 16

GC = 128
TC_BLOCK = 1600


GC0 = 30
GC1 = 10


def _sc_gather(table, idx, e_pad, dt):
  assert e_pad == NS * (GC0 + GC1) * GC
  mesh = plsc.VectorSubcoreMesh(core_axis_name="c", subcore_axis_name="s")

  @functools.partial(
      pl.kernel,
      out_type=jax.ShapeDtypeStruct((e_pad, dt), jnp.float32),
      mesh=mesh,
      scratch_types=[
          pltpu.VMEM((max(GC0, GC1) * GC,), jnp.int32),
          pltpu.VMEM((4, GC, dt), jnp.float32),
          pltpu.SemaphoreType.DMA((4,)),
          pltpu.SemaphoreType.DMA((4,)),
      ],
  )
  def gk(table_hbm, idx_hbm, out_hbm, idxb, rows_v, gsem, wsem):
    cid = lax.axis_index("c")
    sid = lax.axis_index("s")

    def pipeline(base, steps):
      pltpu.sync_copy(idx_hbm.at[pl.ds(base, steps * GC)],
                      idxb.at[pl.ds(0, steps * GC)])
      g, w = {}, {}

      def start_gather(j):
        g[j] = pltpu.async_copy(table_hbm.at[idxb.at[pl.ds(j * GC, GC)]],
                                rows_v.at[j % 4], gsem.at[j % 4])

      for j in range(min(3, steps)):
        start_gather(j)
      for i in range(steps):
        g[i].wait()
        w[i] = pltpu.async_copy(
            rows_v.at[i % 4],
            out_hbm.at[pl.ds(pl.multiple_of(base + i * GC, GC), GC)],
            wsem.at[i % 4])
        nxt = i + 3
        if nxt < steps:
          if nxt - 4 >= 0:
            w[nxt - 4].wait()
          start_gather(nxt)
      for k in range(max(0, steps - 4), steps):
        w[k].wait()

    @pl.when(cid == 0)
    def _core0():
      pipeline(pl.multiple_of(sid * (GC0 * GC), GC), GC0)

    @pl.when(cid == 1)
    def _core1():
      pipeline(pl.multiple_of(NS * (GC0 * GC) + sid * (GC1 * GC), GC), GC1)

  return gk(table, idx)


def _sc_scatter_add(t, dst2, zeros, n_pad, dt):
  npc = n_pad // NC
  acc_rows = npc + 8
  e_pad = t.shape[0]
  steps = e_pad // NS // GC
  rows_per_tile = npc // NS
  mesh = plsc.VectorSubcoreMesh(core_axis_name="c", subcore_axis_name="s")

  @functools.partial(
      pl.kernel,
      out_type=jax.ShapeDtypeStruct((n_pad, dt), jnp.float32),
      mesh=mesh,
      scratch_types=[
          pltpu.VMEM((steps, GC), jnp.int32),
          pltpu.VMEM((steps, GC), jnp.int32),
          pltpu.VMEM((2, GC, dt), jnp.float32),
          pltpu.VMEM((rows_per_tile, dt), jnp.float32),
          pltpu.VMEM_SHARED((acc_rows, dt), jnp.float32),
          pltpu.SemaphoreType.DMA((2,)),
          pltpu.SemaphoreType.DMA((2,)),
      ],
  )
  def sk(t_hbm, dst_hbm, zeros_hbm, out_hbm, di, ri, rows_v, obuf, acc,
         lsem, asem):
    cid = lax.axis_index("c")
    sid = lax.axis_index("s")
    lo = cid * npc

    @pl.when(sid == 0)
    def _init():
      pltpu.sync_copy(zeros_hbm, acc)

    pltpu.sync_copy(dst_hbm.at[sid], di)

    def remap(r0, carry):
      for v in range(GC // LANES):
        dv = di[r0, pl.ds(v * LANES, LANES)]
        local = dv - lo
        ok = (local >= 0) & (local < npc)
        ri[r0, pl.ds(v * LANES, LANES)] = jnp.where(ok, local, npc)
      return carry

    lax.fori_loop(0, steps, remap, 0)
    plsc.subcore_barrier()

    base = sid * steps * GC
    ld, ad = {}, {}

    def start_load(j):
      ld[j] = pltpu.async_copy(
          t_hbm.at[pl.ds(pl.multiple_of(base + j * GC, GC), GC)],
          rows_v.at[j % 2], lsem.at[j % 2])

    for j in range(min(2, steps)):
      start_load(j)
    for i in range(steps):
      ld[i].wait()
      ad[i] = pltpu.async_copy(rows_v.at[i % 2], acc.at[ri.at[i]],
                               asem.at[i % 2], add=True)
      if i + 2 < steps:
        ad[i].wait()
        start_load(i + 2)
    for k in range(max(0, steps - 2), steps):
      ad[k].wait()

    plsc.subcore_barrier()
    pltpu.sync_copy(
        acc.at[pl.ds(pl.multiple_of(sid * rows_per_tile, 8), rows_per_tile)],
        obuf)
    pltpu.sync_copy(
        obuf,
        out_hbm.at[pl.ds(pl.multiple_of(lo + sid * rows_per_tile, 8),
                         rows_per_tile)])

  return sk(t, dst2, zeros)


def _tc_transform(xg, bond, kc, r, s):
  e_pad, dt = xg.shape
  e, bd = bond.shape
  d, gd = kc.shape

  gb = bd * d

  def body(x_ref, c_ref, kc_ref, r_ref, s_ref, o_ref):
    xw = x_ref[...]
    x = xw[:, :d].astype(jnp.bfloat16)
    z = jnp.dot(x, kc_ref[...],
                preferred_element_type=jnp.float32).astype(jnp.bfloat16)
    ce = jnp.dot(c_ref[...], r_ref[...],
                 preferred_element_type=jnp.float32).astype(jnp.bfloat16)
    o_ref[:, :d] = (jnp.dot(z[:, :gb] * ce, s_ref[...],
                            preferred_element_type=jnp.float32)
                    + z[:, gb:gb + d].astype(jnp.float32))

  return pl.pallas_call(
      body,
      grid=(e // TC_BLOCK,),
      in_specs=[
          pl.BlockSpec((TC_BLOCK, dt), lambda i: (i, 0)),
          pl.BlockSpec((TC_BLOCK, bd), lambda i: (i, 0)),
          pl.BlockSpec((d, gd), lambda i: (0, 0)),
          pl.BlockSpec((bd, gb), lambda i: (0, 0)),
          pl.BlockSpec((gb, d), lambda i: (0, 0)),
      ],
      out_specs=pl.BlockSpec((TC_BLOCK, dt), lambda i: (i, 0)),
      out_shape=jax.ShapeDtypeStruct((e_pad, dt), jnp.float32),
      compiler_params=pltpu.CompilerParams(
          dimension_semantics=("arbitrary",)),
  )(xg, bond, kc, r, s)


def kernel(atom_features, bond_features, pair_indices, kernel, bias):
  n, d = atom_features.shape
  e, bd = bond_features.shape
  dp = 128
  groups = bd + 2
  gd = groups * d

  align = NC * NS * GC
  e_pad = -(-e // align) * align
  n_pad = -(-n // (NC * NS * 8)) * (NC * NS * 8)

  src = pair_indices[:, 1].astype(jnp.int32)
  dst = pair_indices[:, 0].astype(jnp.int32)
  pad = e_pad - e
  src = jnp.concatenate([src, jnp.zeros((pad,), jnp.int32)])
  dst = jnp.concatenate([dst, jnp.full((pad,), n, jnp.int32)])
  dst2 = dst.reshape(NS, e_pad // NS // GC, GC)

  k3 = kernel.reshape(bd, d, d).transpose(2, 0, 1)
  b3 = bias.reshape(d, d).T[:, None, :]
  kc = jnp.concatenate([k3, b3, jnp.zeros((d, 1, d), jnp.float32)],
                       axis=1).reshape(d, gd).astype(jnp.bfloat16)
  r = jnp.repeat(jnp.eye(bd, dtype=jnp.float32), d, axis=1)
  s = jnp.tile(jnp.eye(d, dtype=jnp.bfloat16), (bd, 1))

  table = jnp.concatenate(
      [atom_features, jnp.zeros((n, dp - d), jnp.float32)], axis=1)
  gathered = _sc_gather(table, src, e_pad, dp)
  t = _tc_transform(gathered, bond_features, kc, r, s)
  zeros = jnp.zeros((n_pad // NC + 8, dp), jnp.float32)
  out = _sc_scatter_add(t, dst2, zeros, n_pad, dp)
  return out[:n, :d]
